# baseline (speedup 1.0000x reference)
import functools

import jax
import jax.numpy as jnp
from jax import lax
from jax.experimental import pallas as pl
from jax.experimental.pallas import tpu as pltpu
from jax.experimental.pallas import tpu_sc as plsc

N = 50000
D = 256
TBL = 64
L = 16
NC = 2
NS = 16
NW = NC * NS
R = 80
NBLK = N // R
CUNROLL = 32

_mesh = plsc.VectorSubcoreMesh(core_axis_name="c", subcore_axis_name="s")


@functools.partial(
    pl.kernel,
    mesh=_mesh,
    compiler_params=pltpu.CompilerParams(
        use_tc_tiling_on_sc=False, needs_layout_passes=False),
    out_type=jax.ShapeDtypeStruct((N * D,), jnp.float32),
    scratch_types=[
        pltpu.VMEM((R,), jnp.int32),
        pltpu.VMEM((R * D,), jnp.float32),
        pltpu.VMEM((TBL * D,), jnp.float32),
        pltpu.VMEM((TBL * D,), jnp.float32),
        pltpu.SemaphoreType.DMA,
    ],
)
def _sc_add(x_hbm, rank_hbm, zin_hbm, zout_hbm, out_hbm,
            idx_v, xb, zc, zt, sem_x):
    wid = lax.axis_index("s") * NC + lax.axis_index("c")

    pltpu.sync_copy(zin_hbm, zc)
    pltpu.sync_copy(zout_hbm, zt)

    def tstep(t, c2):
        for u in range(16):
            sl = pl.ds((t * 16 + u) * L, L)
            zc[sl] = zc[sl] + zt[sl]
        return c2

    lax.fori_loop(0, TBL * D // (16 * L), tstep, 0)

    lane = lax.iota(jnp.int32, L)

    def blk_body(k, carry):
        b = wid + k * NW
        base = b * R
        pltpu.sync_copy(rank_hbm.at[pl.ds(base, R)], idx_v)
        pltpu.async_copy(x_hbm.at[pl.ds(base * D, R * D)], xb, sem_x).wait()

        def grp_body(j, c2):
            xbase = (lane + j * L) * D
            zbase = idx_v[pl.ds(j * L, L)] * D

            def col_body(cc, c3):
                c0 = cc * CUNROLL
                for u in range(CUNROLL):
                    v = plsc.load_gather(zc, [zbase + (c0 + u)])
                    plsc.addupdate_scatter(xb, [xbase + (c0 + u)], v)
                return c3

            lax.fori_loop(0, D // CUNROLL, col_body, 0)
            return c2

        lax.fori_loop(0, R // L, grp_body, 0)
        pltpu.sync_copy(xb, out_hbm.at[pl.ds(base * D, R * D)])
        return carry

    cnt = (NBLK - 1 - wid) // NW + 1
    lax.fori_loop(0, cnt, blk_body, 0)


def kernel(x, rank, z_in, z_out):
    out = _sc_add(x.reshape(-1), rank.astype(jnp.int32),
                  z_in.reshape(-1), z_out.reshape(-1))
    return out.reshape(N, D)


# speedup vs baseline: 2.6138x; 2.6138x over previous
import functools

import jax
import jax.numpy as jnp
from jax import lax
from jax.experimental import pallas as pl
from jax.experimental.pallas import tpu as pltpu
from jax.experimental.pallas import tpu_sc as plsc

N = 50000
D = 256
TBL = 64
L = 16
NC = 2
NS = 16
NW = NC * NS
R = 80
NBLK = N // R
CUNROLL = 32

_mesh = plsc.VectorSubcoreMesh(core_axis_name="c", subcore_axis_name="s")

_GATHER_DNUMS = lax.GatherDimensionNumbers(
    offset_dims=(), collapsed_slice_dims=(0,), start_index_map=(0,))


def _lane_broadcast(vec, l):
    idx = (lax.iota(jnp.int32, L) * 0 + l)[:, None]
    return lax.gather(vec, idx, _GATHER_DNUMS, slice_sizes=(1,),
                      mode=lax.GatherScatterMode.PROMISE_IN_BOUNDS)


@functools.partial(
    pl.kernel,
    mesh=_mesh,
    compiler_params=pltpu.CompilerParams(
        use_tc_tiling_on_sc=False, needs_layout_passes=False),
    out_type=jax.ShapeDtypeStruct((N * D,), jnp.float32),
    scratch_types=[
        pltpu.VMEM((R,), jnp.int32),
        pltpu.VMEM((R * D,), jnp.float32),
        pltpu.VMEM((TBL * D,), jnp.float32),
        pltpu.VMEM((TBL * D,), jnp.float32),
        pltpu.SemaphoreType.DMA,
    ],
)
def _sc_add(x_hbm, rank_hbm, zin_hbm, zout_hbm, out_hbm,
            idx_v, xb, zc, zt, sem_x):
    wid = lax.axis_index("s") * NC + lax.axis_index("c")

    pltpu.sync_copy(zin_hbm, zc)
    pltpu.sync_copy(zout_hbm, zt)

    def tstep(t, c2):
        for u in range(16):
            sl = pl.ds((t * 16 + u) * L, L)
            zc[sl] = zc[sl] + zt[sl]
        return c2

    lax.fori_loop(0, TBL * D // (16 * L), tstep, 0)

    lane = lax.iota(jnp.int32, L)

    def blk_body(k, carry):
        b = wid + k * NW
        base = b * R
        pltpu.sync_copy(rank_hbm.at[pl.ds(base, R)], idx_v)
        pltpu.async_copy(x_hbm.at[pl.ds(base * D, R * D)], xb, sem_x).wait()

        def grp_body(j, c2):
            rv = idx_v[pl.ds(j * L, L)]
            for l in range(L):
                ri = _lane_broadcast(rv, l)
                zrow = ri * D + lane
                xoff = (j * L + l) * D
                for c in range(D // L):
                    v = plsc.load_gather(zc, [zrow + c * L])
                    plsc.addupdate(xb.at[pl.ds(xoff + c * L, L)], v)
            return c2

        lax.fori_loop(0, R // L, grp_body, 0)
        pltpu.sync_copy(xb, out_hbm.at[pl.ds(base * D, R * D)])
        return carry

    cnt = (NBLK - 1 - wid) // NW + 1
    lax.fori_loop(0, cnt, blk_body, 0)


def kernel(x, rank, z_in, z_out):
    out = _sc_add(x.reshape(-1), rank.astype(jnp.int32),
                  z_in.reshape(-1), z_out.reshape(-1))
    return out.reshape(N, D)


# speedup vs baseline: 5.4458x; 2.0835x over previous
import functools

import jax
import jax.numpy as jnp
from jax import lax
from jax.experimental import pallas as pl
from jax.experimental.pallas import tpu as pltpu
from jax.experimental.pallas import tpu_sc as plsc

N = 50000
D = 256
TBL = 64
L = 16
NC = 2
NS = 16
NW = NC * NS
R = 80
NBLK = N // R
KMAIN = 19
MAIN = NW * KMAIN
TAIL = NBLK - MAIN

_mesh = plsc.VectorSubcoreMesh(core_axis_name="c", subcore_axis_name="s")


def _combine_tables(z_in, z_out):
    def body(a_ref, b_ref, o_ref):
        o_ref[...] = a_ref[...] + b_ref[...]

    return pl.pallas_call(
        body,
        out_shape=jax.ShapeDtypeStruct((TBL, D), jnp.float32),
    )(z_in, z_out)


@functools.partial(
    pl.kernel,
    mesh=_mesh,
    out_type=jax.ShapeDtypeStruct((N, D), jnp.float32),
    scratch_types=[
        pltpu.VMEM((KMAIN * R,), jnp.int32),
        pltpu.VMEM((R,), jnp.int32),
        pltpu.VMEM((R, D), jnp.float32),
        pltpu.VMEM((R, D), jnp.float32),
        pltpu.VMEM((R, D), jnp.float32),
        pltpu.VMEM((R, D), jnp.float32),
        pltpu.SemaphoreType.DMA,
        pltpu.SemaphoreType.DMA,
        pltpu.SemaphoreType.DMA,
        pltpu.SemaphoreType.DMA,
        pltpu.SemaphoreType.DMA,
        pltpu.SemaphoreType.DMA,
    ],
)
def _sc_add(x_hbm, rank_hbm, zc_hbm, out_hbm,
            idx_all, idx_t, xb0, xb1, zb0, zb1,
            sem_z0, sem_z1, sem_x0, sem_x1, sem_o0, sem_o1):
    cid = lax.axis_index("c")
    sid = lax.axis_index("s")
    wid = sid * NC + cid

    xbufs = (xb0, xb1)
    zbufs = (zb0, zb1)
    semz = (sem_z0, sem_z1)
    semx = (sem_x0, sem_x1)
    semo = (sem_o0, sem_o1)

    s0 = wid * KMAIN
    pltpu.sync_copy(rank_hbm.at[pl.ds(s0 * R, KMAIN * R)], idx_all)

    def fire_in(k, slot):
        pltpu.async_copy(zc_hbm.at[idx_all.at[pl.ds(k * R, R)]], zbufs[slot], semz[slot])
        pltpu.async_copy(x_hbm.at[pl.ds((s0 + k) * R, R)], xbufs[slot],
                         semx[slot])

    def wait_in(k, slot):
        pltpu.make_async_copy(zc_hbm.at[idx_all.at[pl.ds(k * R, R)]], zbufs[slot],
                              semz[slot]).wait()
        pltpu.make_async_copy(x_hbm.at[pl.ds((s0 + k) * R, R)], xbufs[slot],
                              semx[slot]).wait()

    def fire_out(k, slot):
        pltpu.async_copy(xbufs[slot], out_hbm.at[pl.ds((s0 + k) * R, R)],
                         semo[slot])

    def wait_out(k, slot):
        pltpu.make_async_copy(xbufs[slot], out_hbm.at[pl.ds((s0 + k) * R, R)],
                              semo[slot]).wait()

    def add_block(xb, zb):
        def row(i, c2):
            for c in range(D // L):
                sl = pl.ds(c * L, L)
                xb[i, sl] = xb[i, sl] + zb[i, sl]
            return c2

        lax.fori_loop(0, R, row, 0)

    fire_in(0, 0)

    def pair_body(k2, carry):
        for u in (0, 1):
            s, sp = u, 1 - u
            k = k2 * 2 + u

            @pl.when((k >= 1) & (k <= KMAIN))
            def _drain_prev():
                wait_out(k - 1, sp)

            @pl.when(k + 1 < KMAIN)
            def _prefetch():
                fire_in(k + 1, sp)

            @pl.when(k < KMAIN)
            def _process():
                wait_in(k, s)
                add_block(xbufs[s], zbufs[s])
                fire_out(k, s)

        return carry

    lax.fori_loop(0, (KMAIN + 2) // 2, pair_body, 0)

    @pl.when(wid < TAIL)
    def _tail():
        tb = MAIN + wid
        pltpu.sync_copy(rank_hbm.at[pl.ds(tb * R, R)], idx_t)
        pltpu.async_copy(zc_hbm.at[idx_t], zb1, sem_z1)
        pltpu.async_copy(x_hbm.at[pl.ds(tb * R, R)], xb1, sem_x1)
        pltpu.make_async_copy(zc_hbm.at[idx_t], zb1, sem_z1).wait()
        pltpu.make_async_copy(x_hbm.at[pl.ds(tb * R, R)], xb1, sem_x1).wait()
        add_block(xb1, zb1)
        pltpu.sync_copy(xb1, out_hbm.at[pl.ds(tb * R, R)])


def kernel(x, rank, z_in, z_out):
    zc = _combine_tables(z_in, z_out)
    return _sc_add(x, rank.astype(jnp.int32), zc)
